# baseline (speedup 1.0000x reference)
import jax
import jax.numpy as jnp
from jax import lax
from jax.experimental import pallas as pl
from jax.experimental.pallas import tpu as pltpu

N = 5000
NP = 5632
R = 44
RB = 40
CH = 4
C = 128
IOU_T = 0.5
SCORE_T = 0.05


def _nms_body(x1_ref, y1_ref, x2_ref, y2_ref, s_ref, keep_ref, area_ref):
    area_ref[:] = (jnp.maximum(x2_ref[:] - x1_ref[:], 0.0)
                   * jnp.maximum(y2_ref[:] - y1_ref[:], 0.0))
    keep_ref[:] = (s_ref[:] > SCORE_T).astype(jnp.float32)

    ii = lax.broadcasted_iota(jnp.int32, (C, C), 0)
    jj = lax.broadcasted_iota(jnp.int32, (C, C), 1)
    diag = (ii == jj).astype(jnp.float32)
    tri = (ii < jj).astype(jnp.float32)

    def row_slices(c):
        return (x1_ref[pl.ds(c, 1), :], y1_ref[pl.ds(c, 1), :],
                x2_ref[pl.ds(c, 1), :], y2_ref[pl.ds(c, 1), :],
                area_ref[pl.ds(c, 1), :])

    def to_col(v_row):
        return jnp.sum(jnp.broadcast_to(v_row, (C, C)) * diag, axis=1,
                       keepdims=True)

    def iou_gt(cols, rows):
        xb1, yb1, xb2, yb2, ab = cols
        xr1, yr1, xr2, yr2, ar = rows
        xx1 = jnp.maximum(xb1, xr1)
        yy1 = jnp.maximum(yb1, yr1)
        xx2 = jnp.minimum(xb2, xr2)
        yy2 = jnp.minimum(yb2, yr2)
        inter = jnp.maximum(xx2 - xx1, 0.0) * jnp.maximum(yy2 - yy1, 0.0)
        union = ab + ar - inter
        iou = inter / (union + 1e-9)
        return (iou > IOU_T).astype(jnp.float32)

    def outer(r, _):
        rows_r = row_slices(r)
        cols_r = tuple(to_col(v) for v in rows_r)
        m_intra = iou_gt(cols_r, rows_r) * tri

        valid = keep_ref[pl.ds(r, 1), :]

        def f_cond(st):
            return st[1]

        def f_body(st):
            kb, _ = st
            supp = lax.dot_general(kb, m_intra, (((1,), (0,)), ((), ())),
                                   preferred_element_type=jnp.float32)
            kb2 = valid * (supp < 0.5).astype(jnp.float32)
            changed = jnp.sum(jnp.abs(kb2 - kb)) > 0.0
            return kb2, changed
        kb, _ = lax.while_loop(f_cond, f_body, (valid, jnp.bool_(True)))
        keep_ref[pl.ds(r, 1), :] = kb

        def inner(k, _):
            c0 = r + 1 + k * CH
            for m in range(CH):
                c = c0 + m
                m_rc = iou_gt(cols_r, row_slices(c))
                supp = lax.dot_general(kb, m_rc, (((1,), (0,)), ((), ())),
                                       preferred_element_type=jnp.float32)
                keep_ref[pl.ds(c, 1), :] = (keep_ref[pl.ds(c, 1), :]
                                            * (supp < 0.5).astype(jnp.float32))
            return 0

        nchunks = (RB - r - 1 + CH - 1) // CH
        return lax.fori_loop(0, nchunks, inner, 0)

    lax.fori_loop(0, RB, outer, 0)


def _nms_keep_sorted(x1, y1, x2, y2, s, interpret=False):
    return pl.pallas_call(
        _nms_body,
        out_shape=jax.ShapeDtypeStruct((R, C), jnp.float32),
        scratch_shapes=[pltpu.VMEM((R, C), jnp.float32)],
        interpret=interpret,
    )(x1, y1, x2, y2, s)


def kernel(y_pred):
    scores = y_pred[:, 4]
    order = jnp.argsort(-scores)
    sb = y_pred[order]
    pad = jnp.concatenate(
        [jnp.zeros((NP - N, 4), jnp.float32),
         jnp.full((NP - N, 1), -1.0, jnp.float32)], axis=1)
    sbp = jnp.concatenate([sb, pad], axis=0)
    cols = [sbp[:, k].reshape(R, C) for k in range(5)]
    keep_s = _nms_keep_sorted(*cols)
    keep_flat = keep_s.reshape(NP)[:N]
    mask = jnp.zeros((N,), jnp.float32).at[order].set(keep_flat)
    return y_pred * mask[:, None]


# speedup vs baseline: 1.0979x; 1.0979x over previous
import jax
import jax.numpy as jnp
from jax import lax
from jax.experimental import pallas as pl
from jax.experimental.pallas import tpu as pltpu

N = 5000
NP = 6144
R = 48
RB = 40
CH = 8
C = 128
IOU_T = 0.5
SCORE_T = 0.05


def _nms_body(x1_ref, y1_ref, x2_ref, y2_ref, s_ref, keep_ref, area_ref):
    area_ref[:] = (jnp.maximum(x2_ref[:] - x1_ref[:], 0.0)
                   * jnp.maximum(y2_ref[:] - y1_ref[:], 0.0))
    keep_ref[:] = (s_ref[:] > SCORE_T).astype(jnp.float32)

    ii = lax.broadcasted_iota(jnp.int32, (C, C), 0)
    jj = lax.broadcasted_iota(jnp.int32, (C, C), 1)
    diag = (ii == jj).astype(jnp.float32)
    tri = (ii < jj).astype(jnp.float32)

    def row_slices(c):
        return (x1_ref[pl.ds(c, 1), :], y1_ref[pl.ds(c, 1), :],
                x2_ref[pl.ds(c, 1), :], y2_ref[pl.ds(c, 1), :],
                area_ref[pl.ds(c, 1), :])

    def to_col(v_row):
        return jnp.sum(jnp.broadcast_to(v_row, (C, C)) * diag, axis=1,
                       keepdims=True)

    def iou_gt(cols, rows):
        xb1, yb1, xb2, yb2, ab = cols
        xr1, yr1, xr2, yr2, ar = rows
        xx1 = jnp.maximum(xb1, xr1)
        yy1 = jnp.maximum(yb1, yr1)
        xx2 = jnp.minimum(xb2, xr2)
        yy2 = jnp.minimum(yb2, yr2)
        inter = jnp.maximum(xx2 - xx1, 0.0) * jnp.maximum(yy2 - yy1, 0.0)
        union = ab + ar - inter
        iou = inter / (union + 1e-9)
        return (iou > IOU_T).astype(jnp.float32)

    def outer(r, _):
        rows_r = row_slices(r)
        cols_r = tuple(to_col(v) for v in rows_r)
        m_intra = iou_gt(cols_r, rows_r) * tri

        valid = keep_ref[pl.ds(r, 1), :]

        def f_cond(st):
            return st[1]

        def f_body(st):
            kb, _ = st
            supp = lax.dot_general(kb, m_intra, (((1,), (0,)), ((), ())),
                                   preferred_element_type=jnp.float32)
            kb2 = valid * (supp < 0.5).astype(jnp.float32)
            changed = jnp.sum(jnp.abs(kb2 - kb)) > 0.0
            return kb2, changed
        kb, _ = lax.while_loop(f_cond, f_body, (valid, jnp.bool_(True)))
        keep_ref[pl.ds(r, 1), :] = kb

        def inner(k, _):
            c0 = r + 1 + k * CH
            for m in range(CH):
                c = c0 + m
                m_rc = iou_gt(cols_r, row_slices(c))
                supp = lax.dot_general(kb, m_rc, (((1,), (0,)), ((), ())),
                                       preferred_element_type=jnp.float32)
                keep_ref[pl.ds(c, 1), :] = (keep_ref[pl.ds(c, 1), :]
                                            * (supp < 0.5).astype(jnp.float32))
            return 0

        nchunks = (RB - r - 1 + CH - 1) // CH
        return lax.fori_loop(0, nchunks, inner, 0)

    lax.fori_loop(0, RB, outer, 0)


def _nms_keep_sorted(x1, y1, x2, y2, s, interpret=False):
    return pl.pallas_call(
        _nms_body,
        out_shape=jax.ShapeDtypeStruct((R, C), jnp.float32),
        scratch_shapes=[pltpu.VMEM((R, C), jnp.float32)],
        interpret=interpret,
    )(x1, y1, x2, y2, s)


def kernel(y_pred):
    scores = y_pred[:, 4]
    order = jnp.argsort(-scores)
    sb = y_pred[order]
    pad = jnp.concatenate(
        [jnp.zeros((NP - N, 4), jnp.float32),
         jnp.full((NP - N, 1), -1.0, jnp.float32)], axis=1)
    sbp = jnp.concatenate([sb, pad], axis=0)
    cols = [sbp[:, k].reshape(R, C) for k in range(5)]
    keep_s = _nms_keep_sorted(*cols)
    keep_flat = keep_s.reshape(NP)[:N]
    mask = jnp.zeros((N,), jnp.float32).at[order].set(keep_flat)
    return y_pred * mask[:, None]
